# Initial kernel scaffold; baseline (speedup 1.0000x reference)
import jax
import jax.numpy as jnp
from jax.experimental import pallas as pl


def kernel(points, Wd, bd, Wa, ba):
    raise NotImplementedError("write your pallas kernel here")


# speedup vs baseline: 2.6490x; 2.6490x over previous
import functools

import numpy as np
import jax
import jax.numpy as jnp
from jax import lax
from jax.experimental import pallas as pl
from jax.experimental.pallas import tpu as pltpu
from jax.experimental.pallas import tpu_sc as plsc

HID = 256
HALF = HID // 2
SIGMA_D = 0.2
SIGMA_A = 15.0
ANGLE_K = 3
TOPK = 35
FACTOR_A = 180.0 / (SIGMA_A * np.pi)
NEG_LOG1E4 = -float(np.log(10000.0))

NUM_WORKERS = 32
GATHER_CHUNKS = 5


def _knn_body(pts_ref, ptsT_ref, idx_ref, sqd_ref, nbrf_ref, r0f_ref,
              r1f_ref, r2f_ref, d2_ref):
  b = pl.program_id(0)
  n = pts_ref.shape[1]
  x = pts_ref[0, :, 0:1]
  y = pts_ref[0, :, 1:2]
  z = pts_ref[0, :, 2:3]
  xT = ptsT_ref[0, 0:1, :]
  yT = ptsT_ref[0, 1:2, :]
  zT = ptsT_ref[0, 2:3, :]
  dx = x - xT
  dy = y - yT
  dz = z - zT
  d2_ref[...] = (dx * dx + dy * dy) + dz * dz
  lane = lax.broadcasted_iota(jnp.int32, (n, n), 1)
  lane_k = lax.broadcasted_iota(jnp.int32, (n, TOPK), 1)

  def body(k, _):
    d2 = d2_ref[...]
    rowmin = jnp.min(d2, axis=1, keepdims=True)
    sel = jnp.min(jnp.where(d2 == rowmin, lane, n), axis=1,
                  keepdims=True)
    d2_ref[...] = jnp.where(lane == sel, jnp.inf, d2)
    at_k = lane_k == k
    idx_ref[0] = jnp.where(at_k, sel, idx_ref[0])
    sqd_ref[0] = jnp.where(at_k, rowmin, sqd_ref[0])
    return 0

  lax.fori_loop(0, TOPK, body, 0)
  base = b * n
  idx = idx_ref[0]
  nbrf_ref[0] = idx + base
  r0f_ref[0] = jnp.broadcast_to(idx[:, 0:1] + base, (n, TOPK))
  r1f_ref[0] = jnp.broadcast_to(idx[:, 1:2] + base, (n, TOPK))
  r2f_ref[0] = jnp.broadcast_to(idx[:, 2:3] + base, (n, TOPK))


def _knn(points):
  bsz, n, _ = points.shape
  pointsT = jnp.swapaxes(points, 1, 2)
  out_shapes = (
      jax.ShapeDtypeStruct((bsz, n, TOPK), jnp.int32),
      jax.ShapeDtypeStruct((bsz, n, TOPK), jnp.float32),
      jax.ShapeDtypeStruct((bsz, n, TOPK), jnp.int32),
      jax.ShapeDtypeStruct((bsz, n, TOPK), jnp.int32),
      jax.ShapeDtypeStruct((bsz, n, TOPK), jnp.int32),
      jax.ShapeDtypeStruct((bsz, n, TOPK), jnp.int32),
  )
  out_spec = pl.BlockSpec((1, n, TOPK), lambda b: (b, 0, 0))
  return pl.pallas_call(
      _knn_body,
      grid=(bsz,),
      in_specs=[
          pl.BlockSpec((1, n, 3), lambda b: (b, 0, 0)),
          pl.BlockSpec((1, 3, n), lambda b: (b, 0, 0)),
      ],
      out_specs=(out_spec,) * 6,
      out_shape=out_shapes,
      scratch_shapes=[pltpu.VMEM((n, n), jnp.float32)],
  )(points, pointsT)


def _gather_rows(table16, idx_flat):
  total = idx_flat.shape[0]
  per_worker = total // NUM_WORKERS
  chunk = per_worker // GATHER_CHUNKS
  mesh = plsc.VectorSubcoreMesh(core_axis_name="c", subcore_axis_name="s")

  @functools.partial(
      pl.kernel,
      out_type=jax.ShapeDtypeStruct((total, 16), jnp.float32),
      mesh=mesh,
      scratch_types=[
          pltpu.VMEM((chunk,), jnp.int32),
          pltpu.VMEM((chunk, 16), jnp.float32),
          pltpu.SemaphoreType.DMA,
      ],
      compiler_params=pltpu.CompilerParams(use_tc_tiling_on_sc=False),
  )
  def gather_kernel(tbl_hbm, idx_hbm, out_hbm, idx_v, rows_v, sem):
    wid = lax.axis_index("s") * 2 + lax.axis_index("c")
    for c in range(GATHER_CHUNKS):
      base = (wid * GATHER_CHUNKS + c) * chunk
      pltpu.sync_copy(idx_hbm.at[pl.ds(base, chunk)], idx_v)
      pltpu.async_copy(tbl_hbm.at[idx_v], rows_v, sem).wait()
      pltpu.sync_copy(rows_v, out_hbm.at[pl.ds(base, chunk)])

  return gather_kernel(table16, idx_flat)


def _embed_body(nbr_ref, ctr_ref, r0_ref, r1_ref, r2_ref, sqd_ref, wdp_ref,
                wap_ref, bd_ref, ba_ref, out_ref):
  ax = nbr_ref[:, 0:1] - ctr_ref[:, 0:1]
  ay = nbr_ref[:, 1:2] - ctr_ref[:, 1:2]
  az = nbr_ref[:, 2:3] - ctr_ref[:, 2:3]

  jj = lax.broadcasted_iota(jnp.int32, (1, HALF), 1).astype(jnp.float32)
  div = jnp.exp(jj * (2.0 * NEG_LOG1E4 / HID))

  d_ind = jnp.sqrt(jnp.maximum(sqd_ref[:, :], 1e-8)) * (1.0 / SIGMA_D)
  om = d_ind * div
  embd = jnp.concatenate([jnp.sin(om), jnp.cos(om)], axis=1)
  acc = jnp.dot(embd, wdp_ref[:, :], preferred_element_type=jnp.float32)

  amax = None
  for r_ref in (r0_ref, r1_ref, r2_ref):
    rx = r_ref[:, 0:1] - nbr_ref[:, 0:1]
    ry = r_ref[:, 1:2] - nbr_ref[:, 1:2]
    rz = r_ref[:, 2:3] - nbr_ref[:, 2:3]
    cx = ry * az - rz * ay
    cy = rz * ax - rx * az
    cz = rx * ay - ry * ax
    sinv = jnp.sqrt(cx * cx + cy * cy + cz * cz)
    cosv = rx * ax + ry * ay + rz * az
    cosv = jnp.where(cosv == 0.0, 0.0, cosv)
    ang = jnp.arctan2(sinv, cosv) * FACTOR_A
    oma = ang * div
    emba = jnp.concatenate([jnp.sin(oma), jnp.cos(oma)], axis=1)
    e = jnp.dot(emba, wap_ref[:, :], preferred_element_type=jnp.float32)
    amax = e if amax is None else jnp.maximum(amax, e)

  out_ref[:, :] = acc + amax + bd_ref[:, :] + ba_ref[:, :]


def _embed(nbr, ctr, r0, r1, r2, sqd_tok, wdp, wap, bd, ba, tile):
  tok = nbr.shape[0]
  grid = tok // tile
  tok_spec = pl.BlockSpec((tile, 16), lambda i: (i, 0))
  return pl.pallas_call(
      _embed_body,
      grid=(grid,),
      in_specs=[
          tok_spec, tok_spec, tok_spec, tok_spec, tok_spec,
          pl.BlockSpec((tile, 1), lambda i: (i, 0)),
          pl.BlockSpec((HID, HID), lambda i: (0, 0)),
          pl.BlockSpec((HID, HID), lambda i: (0, 0)),
          pl.BlockSpec((1, HID), lambda i: (0, 0)),
          pl.BlockSpec((1, HID), lambda i: (0, 0)),
      ],
      out_specs=pl.BlockSpec((tile, HID), lambda i: (i, 0)),
      out_shape=jax.ShapeDtypeStruct((tok, HID), jnp.float32),
  )(nbr, ctr, r0, r1, r2, sqd_tok, wdp, wap, bd, ba)


def kernel(points, Wd, bd, Wa, ba):
  bsz, n, _ = points.shape
  k = min(TOPK, n)
  tok = bsz * n * k

  knn_idx, sqd, nbrf, r0f, r1f, r2f = _knn(points)

  table16 = jnp.pad(points.reshape(bsz * n, 3), ((0, 0), (0, 13)))
  ctrf = jnp.repeat(jnp.arange(bsz * n, dtype=jnp.int32), k)
  idx_all = jnp.concatenate([
      nbrf.reshape(-1), ctrf, r0f.reshape(-1), r1f.reshape(-1),
      r2f.reshape(-1)
  ])
  rows = _gather_rows(table16, idx_all)
  nbr = rows[0 * tok:1 * tok]
  ctr = rows[1 * tok:2 * tok]
  rf0 = rows[2 * tok:3 * tok]
  rf1 = rows[3 * tok:4 * tok]
  rf2 = rows[4 * tok:5 * tok]

  wdp = jnp.concatenate([Wd[:, 0::2], Wd[:, 1::2]], axis=1).T
  wap = jnp.concatenate([Wa[:, 0::2], Wa[:, 1::2]], axis=1).T

  emb = _embed(nbr, ctr, rf0, rf1, rf2, sqd.reshape(tok, 1), wdp, wap,
               bd.reshape(1, HID), ba.reshape(1, HID), tile=1024)
  return emb.reshape(bsz, n, k, HID), knn_idx


# speedup vs baseline: 3.6487x; 1.3774x over previous
import functools

import numpy as np
import jax
import jax.numpy as jnp
from jax import lax
from jax.experimental import pallas as pl
from jax.experimental.pallas import tpu as pltpu
from jax.experimental.pallas import tpu_sc as plsc

HID = 256
HALF = HID // 2
SIGMA_D = 0.2
SIGMA_A = 15.0
ANGLE_K = 3
TOPK = 35
FACTOR_A = 180.0 / (SIGMA_A * np.pi)
NEG_LOG1E4 = -float(np.log(10000.0))

NUM_WORKERS = 32
GATHER_CHUNKS = 5


def _knn_body(pts_ref, ptsT_ref, idx_ref, sqd_ref, nbrf_ref, r0f_ref,
              r1f_ref, r2f_ref, d2_ref):
  b = pl.program_id(0)
  n = pts_ref.shape[1]
  x = pts_ref[0, :, 0:1]
  y = pts_ref[0, :, 1:2]
  z = pts_ref[0, :, 2:3]
  xT = ptsT_ref[0, 0:1, :]
  yT = ptsT_ref[0, 1:2, :]
  zT = ptsT_ref[0, 2:3, :]
  dx = x - xT
  dy = y - yT
  dz = z - zT
  d2_ref[...] = (dx * dx + dy * dy) + dz * dz
  lane = lax.broadcasted_iota(jnp.int32, (n, n), 1)
  lane_k = lax.broadcasted_iota(jnp.int32, (n, TOPK), 1)

  def body(k, _):
    d2 = d2_ref[...]
    rowmin = jnp.min(d2, axis=1, keepdims=True)
    sel = jnp.min(jnp.where(d2 == rowmin, lane, n), axis=1,
                  keepdims=True)
    d2_ref[...] = jnp.where(lane == sel, jnp.inf, d2)
    at_k = lane_k == k
    idx_ref[0] = jnp.where(at_k, sel, idx_ref[0])
    sqd_ref[0] = jnp.where(at_k, rowmin, sqd_ref[0])
    return 0

  lax.fori_loop(0, TOPK, body, 0)
  base = b * n
  idx = idx_ref[0]
  nbrf_ref[0] = idx + base
  r0f_ref[0] = jnp.broadcast_to(idx[:, 0:1] + base, (n, TOPK))
  r1f_ref[0] = jnp.broadcast_to(idx[:, 1:2] + base, (n, TOPK))
  r2f_ref[0] = jnp.broadcast_to(idx[:, 2:3] + base, (n, TOPK))


def _knn(points):
  bsz, n, _ = points.shape
  pointsT = jnp.swapaxes(points, 1, 2)
  out_shapes = (
      jax.ShapeDtypeStruct((bsz, n, TOPK), jnp.int32),
      jax.ShapeDtypeStruct((bsz, n, TOPK), jnp.float32),
      jax.ShapeDtypeStruct((bsz, n, TOPK), jnp.int32),
      jax.ShapeDtypeStruct((bsz, n, TOPK), jnp.int32),
      jax.ShapeDtypeStruct((bsz, n, TOPK), jnp.int32),
      jax.ShapeDtypeStruct((bsz, n, TOPK), jnp.int32),
  )
  out_spec = pl.BlockSpec((1, n, TOPK), lambda b: (b, 0, 0))
  return pl.pallas_call(
      _knn_body,
      grid=(bsz,),
      in_specs=[
          pl.BlockSpec((1, n, 3), lambda b: (b, 0, 0)),
          pl.BlockSpec((1, 3, n), lambda b: (b, 0, 0)),
      ],
      out_specs=(out_spec,) * 6,
      out_shape=out_shapes,
      scratch_shapes=[pltpu.VMEM((n, n), jnp.float32)],
  )(points, pointsT)


def _gather_rows(table16, idx_flat):
  total = idx_flat.shape[0]
  per_worker = total // NUM_WORKERS
  chunk = per_worker // GATHER_CHUNKS
  mesh = plsc.VectorSubcoreMesh(core_axis_name="c", subcore_axis_name="s")

  @functools.partial(
      pl.kernel,
      out_type=jax.ShapeDtypeStruct((total, 16), jnp.float32),
      mesh=mesh,
      scratch_types=[
          pltpu.VMEM((chunk,), jnp.int32),
          pltpu.VMEM((chunk, 16), jnp.float32),
          pltpu.SemaphoreType.DMA,
      ],
      compiler_params=pltpu.CompilerParams(use_tc_tiling_on_sc=False),
  )
  def gather_kernel(tbl_hbm, idx_hbm, out_hbm, idx_v, rows_v, sem):
    wid = lax.axis_index("s") * 2 + lax.axis_index("c")
    for c in range(GATHER_CHUNKS):
      base = (wid * GATHER_CHUNKS + c) * chunk
      pltpu.sync_copy(idx_hbm.at[pl.ds(base, chunk)], idx_v)
      pltpu.async_copy(tbl_hbm.at[idx_v], rows_v, sem).wait()
      pltpu.sync_copy(rows_v, out_hbm.at[pl.ds(base, chunk)])

  return gather_kernel(table16, idx_flat)


_INV_2PI = float(1.0 / (2.0 * np.pi))
_SIN_C = (6.283088463027395, -41.333247542218885, 81.40008976706689,
          -74.6758838695101, 33.16809461334915)
_COS_C = (0.9999994434155783, -19.739034322006077, 64.93061147431379,
          -85.29594600637847, 58.912422344014445, -21.282776325506184)


def _sincos(t):
  r = t * _INV_2PI
  r = r - jnp.round(r)
  s2 = r * r
  s = jnp.float32(_SIN_C[-1])
  for a in _SIN_C[-2::-1]:
    s = s * s2 + jnp.float32(a)
  s = s * r
  c = jnp.float32(_COS_C[-1])
  for a in _COS_C[-2::-1]:
    c = c * s2 + jnp.float32(a)
  return s, c


def _embed_body(nbr_ref, ctr_ref, r0_ref, r1_ref, r2_ref, sqd_ref, wdp_ref,
                wap_ref, bd_ref, ba_ref, out_ref):
  ax = nbr_ref[:, 0:1] - ctr_ref[:, 0:1]
  ay = nbr_ref[:, 1:2] - ctr_ref[:, 1:2]
  az = nbr_ref[:, 2:3] - ctr_ref[:, 2:3]

  jj = lax.broadcasted_iota(jnp.int32, (1, HALF), 1).astype(jnp.float32)
  div = jnp.exp(jj * (2.0 * NEG_LOG1E4 / HID))

  d_ind = jnp.sqrt(jnp.maximum(sqd_ref[:, :], 1e-8)) * (1.0 / SIGMA_D)
  om = d_ind * div
  s_d, c_d = _sincos(om)
  embd = jnp.concatenate([s_d, c_d], axis=1)
  acc = jnp.dot(embd, wdp_ref[:, :], preferred_element_type=jnp.float32)

  amax = None
  for r_ref in (r0_ref, r1_ref, r2_ref):
    rx = r_ref[:, 0:1] - nbr_ref[:, 0:1]
    ry = r_ref[:, 1:2] - nbr_ref[:, 1:2]
    rz = r_ref[:, 2:3] - nbr_ref[:, 2:3]
    cx = ry * az - rz * ay
    cy = rz * ax - rx * az
    cz = rx * ay - ry * ax
    sinv = jnp.sqrt(cx * cx + cy * cy + cz * cz)
    cosv = rx * ax + ry * ay + rz * az
    cosv = jnp.where(cosv == 0.0, 0.0, cosv)
    ang = jnp.arctan2(sinv, cosv) * FACTOR_A
    oma = ang * div
    s_a, c_a = _sincos(oma)
    emba = jnp.concatenate([s_a, c_a], axis=1)
    e = jnp.dot(emba, wap_ref[:, :], preferred_element_type=jnp.float32)
    amax = e if amax is None else jnp.maximum(amax, e)

  out_ref[:, :] = acc + amax + bd_ref[:, :] + ba_ref[:, :]


def _embed(nbr, ctr, r0, r1, r2, sqd_tok, wdp, wap, bd, ba, tile):
  tok = nbr.shape[0]
  grid = tok // tile
  tok_spec = pl.BlockSpec((tile, 16), lambda i: (i, 0))
  return pl.pallas_call(
      _embed_body,
      grid=(grid,),
      in_specs=[
          tok_spec, tok_spec, tok_spec, tok_spec, tok_spec,
          pl.BlockSpec((tile, 1), lambda i: (i, 0)),
          pl.BlockSpec((HID, HID), lambda i: (0, 0)),
          pl.BlockSpec((HID, HID), lambda i: (0, 0)),
          pl.BlockSpec((1, HID), lambda i: (0, 0)),
          pl.BlockSpec((1, HID), lambda i: (0, 0)),
      ],
      out_specs=pl.BlockSpec((tile, HID), lambda i: (i, 0)),
      out_shape=jax.ShapeDtypeStruct((tok, HID), jnp.float32),
  )(nbr, ctr, r0, r1, r2, sqd_tok, wdp, wap, bd, ba)


def kernel(points, Wd, bd, Wa, ba):
  bsz, n, _ = points.shape
  k = min(TOPK, n)
  tok = bsz * n * k

  knn_idx, sqd, nbrf, r0f, r1f, r2f = _knn(points)

  table16 = jnp.pad(points.reshape(bsz * n, 3), ((0, 0), (0, 13)))
  ctrf = jnp.repeat(jnp.arange(bsz * n, dtype=jnp.int32), k)
  idx_all = jnp.concatenate([
      nbrf.reshape(-1), ctrf, r0f.reshape(-1), r1f.reshape(-1),
      r2f.reshape(-1)
  ])
  rows = _gather_rows(table16, idx_all)
  nbr = rows[0 * tok:1 * tok]
  ctr = rows[1 * tok:2 * tok]
  rf0 = rows[2 * tok:3 * tok]
  rf1 = rows[3 * tok:4 * tok]
  rf2 = rows[4 * tok:5 * tok]

  wdp = jnp.concatenate([Wd[:, 0::2], Wd[:, 1::2]], axis=1).T
  wap = jnp.concatenate([Wa[:, 0::2], Wa[:, 1::2]], axis=1).T

  emb = _embed(nbr, ctr, rf0, rf1, rf2, sqd.reshape(tok, 1), wdp, wap,
               bd.reshape(1, HID), ba.reshape(1, HID), tile=1024)
  return emb.reshape(bsz, n, k, HID), knn_idx
